# baseline (speedup 1.0000x reference)
import jax
import jax.numpy as jnp
from jax import lax
from jax.experimental import pallas as pl
from jax.experimental.pallas import tpu as pltpu


def _mm_kernel(ids_ref, x_ref, w_ref, o_ref):
    acc = lax.dot_general(
        w_ref[0], x_ref[0],
        dimension_numbers=(((0,), (1,)), ((), ())),
        preferred_element_type=jnp.float32,
    )
    o_ref[0] = acc.T


def kernel(x, weight, weight_active, adapter_ids, seq_ids):
    B, S, D = x.shape
    R = weight.shape[-1]
    grid_spec = pltpu.PrefetchScalarGridSpec(
        num_scalar_prefetch=1,
        grid=(B,),
        in_specs=[
            pl.BlockSpec((1, S, D), lambda b, ids: (b, 0, 0)),
            pl.BlockSpec((1, D, R), lambda b, ids: (ids[b], 0, 0)),
        ],
        out_specs=pl.BlockSpec((1, S, R), lambda b, ids: (b, 0, 0)),
    )
    return pl.pallas_call(
        _mm_kernel,
        grid_spec=grid_spec,
        out_shape=jax.ShapeDtypeStruct((B, S, R), x.dtype),
    )(adapter_ids.astype(jnp.int32), x, weight)
